# baseline (speedup 1.0000x reference)
import functools

import jax
import jax.numpy as jnp
from jax import lax
from jax.experimental import pallas as pl
from jax.experimental.pallas import tpu as pltpu
from jax.experimental.pallas import tpu_sc as plsc

VOCAB = 1000000
EMBED = 64
NCLS = 5
BATCH = 16384
HIST = 200

DP = 8
NC, NS = 2, 16
NW = NC * NS
SPW = BATCH // NW
CS = 16
NCHUNK = SPW // CS
G0, G1 = 104, 96


RB = 32768
SLOTS = 128 // DP
SB = RB // SLOTS
SHIFT = 11
NBLK = -(-VOCAB // RB)
VP = NBLK * RB


def _proj_body(et_ref, w_ref, p_ref):
    lhs = jnp.concatenate(
        [et_ref[:, SB * m:SB * (m + 1)] for m in range(SLOTS)], axis=0)
    p_ref[...] = lax.dot_general(lhs, w_ref[...],
                                 dimension_numbers=(((0,), (0,)), ((), ())),
                                 preferred_element_type=jnp.float32)


def _project_table(embed_t, w_pad):
    return pl.pallas_call(
        _proj_body,
        grid=(NBLK,),
        in_specs=[
            pl.BlockSpec((EMBED, RB), lambda i: (0, i)),
            pl.BlockSpec((SLOTS * EMBED, 128), lambda i: (0, 0)),
        ],
        out_specs=pl.BlockSpec((RB // SLOTS, 128), lambda i: (i, 0)),
        out_shape=jax.ShapeDtypeStruct((VP // SLOTS, 128), jnp.float32),
    )(embed_t, w_pad)


def _sc_body(p_hbm, idx_hbm, b_hbm, out_hbm,
             idxtb0, idxtb1, idxb0, idxb1, rowsb0, rowsb1, outv, bvecb, foldb,
             semg0, semg1, semi0, semi1):
    wid = lax.axis_index("s") * NC + lax.axis_index("c")
    base = wid * SPW
    pltpu.sync_copy(b_hbm, bvecb)
    foldb[pl.ds(8, 16)] = jnp.zeros((16,), jnp.float32)
    bufs = ((idxtb0, idxb0, rowsb0, semg0, semi0),
            (idxtb1, idxb1, rowsb1, semg1, semi1))

    def stage_idx(c, idxtb, semi):
        s0 = base + c * CS
        a, b0 = s0 // 128, s0 % 128
        for t in range(HIST // 8):
            pltpu.async_copy(idx_hbm.at[t, a, :, pl.ds(b0, CS)],
                             idxtb.at[t], semi)

    def wait_idx(c, idxtb, semi):
        s0 = base + c * CS
        pltpu.make_async_copy(
            idx_hbm.at[:, s0 // 128, :, pl.ds(s0 % 128, CS)],
            idxtb, semi).wait()

    def remap(idxtb, idxb):
        hi3 = lax.iota(jnp.int32, 16) >> 3
        lo3 = lax.iota(jnp.int32, 16) & 7
        for s in range(CS):
            sv = jnp.full((16,), s, jnp.int32)
            for t in range(13):
                col = min(16 * t, HIST - 16)
                vv = plsc.load_gather(idxtb, [hi3 + (col >> 3), lo3, sv])
                idxb[pl.ds(HIST * s + col, 16)] = (
                    (vv & jnp.int32(~(RB - 1)))
                    | ((vv & jnp.int32(SB - 1)) << 4)
                    | ((vv >> SHIFT) & (SLOTS - 1)))

    def fire_gathers(idxb, rowsb, semg):
        for k in range(CS * HIST // 128):
            pltpu.async_copy(p_hbm.at[idxb.at[pl.ds(128 * k, 128)]],
                             rowsb.at[pl.ds(128 * k, 128)], semg)

    def drain_gathers(rowsb, semg):
        pltpu.make_async_copy(p_hbm.at[pl.ds(0, CS * HIST)], rowsb, semg).wait()

    def accumulate(c, rowsb):
        bv = bvecb[...]
        z = jnp.zeros((16,), jnp.float32)
        rowpat = (lax.iota(jnp.int32, 16) >> 3) & 1
        colpat = lax.iota(jnp.int32, 16) & 7
        for s in range(CS):
            def body(j, accs):
                r = HIST * s + 8 * j
                return tuple(
                    accs[t] + plsc.load_gather(
                        rowsb, [rowpat + (r + 2 * t), colpat])
                    for t in range(4))
            accs = lax.fori_loop(0, HIST // 8, body, (z,) * 4)
            tot = (accs[0] + accs[1]) + (accs[2] + accs[3])
            foldb[pl.ds(0, 16)] = tot
            outv[c * CS + s] = tot + foldb[pl.ds(8, 16)] + bv

    idxtb, idxb, rowsb, semg, semi = bufs[0]
    stage_idx(0, idxtb, semi)
    wait_idx(0, idxtb, semi)
    remap(idxtb, idxb)
    fire_gathers(idxb, rowsb, semg)
    stage_idx(1, bufs[1][0], bufs[1][4])
    wait_idx(1, bufs[1][0], bufs[1][4])
    remap(bufs[1][0], bufs[1][1])

    def half(c, b):
        idxtb, idxb, rowsb, semg, semi = bufs[b]
        nidxtb, nidxb, nrowsb, nsemg, nsemi = bufs[1 - b]
        drain_gathers(rowsb, semg)

        @pl.when(c + 1 < NCHUNK)
        def _():
            fire_gathers(nidxb, nrowsb, nsemg)

        @pl.when(c + 2 < NCHUNK)
        def _():
            stage_idx(c + 2, idxtb, semi)

        accumulate(c, rowsb)

        @pl.when(c + 2 < NCHUNK)
        def _():
            wait_idx(c + 2, idxtb, semi)
            remap(idxtb, idxb)

    def pair(cc, carry):
        half(2 * cc, 0)
        half(2 * cc + 1, 1)
        return carry

    lax.fori_loop(0, NCHUNK // 2, pair, 0)
    pltpu.sync_copy(outv, out_hbm.at[pl.ds(base, SPW)])


@jax.jit
def kernel(inputs, embed_table, W, b):
    b_pad = jnp.zeros((16,), jnp.float32).at[:NCLS].set(b)
    w_bd = jnp.zeros((SLOTS * EMBED, 128), jnp.float32)
    for _m in range(SLOTS):
        w_bd = w_bd.at[EMBED * _m:EMBED * (_m + 1),
                       DP * _m:DP * _m + NCLS].set(W)
    p = jnp.reshape(_project_table(embed_table.T, w_bd), (VP, DP))
    idx4 = jnp.transpose(jnp.reshape(
        inputs.astype(jnp.int32),
        (BATCH // 128, 128, HIST // 8, 8)), (2, 0, 3, 1))

    mesh = plsc.VectorSubcoreMesh(core_axis_name="c", subcore_axis_name="s")
    out16 = pl.kernel(
        _sc_body,
        out_type=jax.ShapeDtypeStruct((BATCH, 16), jnp.float32),
        mesh=mesh,
        compiler_params=pltpu.CompilerParams(use_tc_tiling_on_sc=False,
                                             needs_layout_passes=False),
        scratch_types=[
            pltpu.VMEM((HIST // 8, 8, CS), jnp.int32),
            pltpu.VMEM((HIST // 8, 8, CS), jnp.int32),
            pltpu.VMEM((CS * HIST,), jnp.int32),
            pltpu.VMEM((CS * HIST,), jnp.int32),
            pltpu.VMEM((CS * HIST, DP), jnp.float32),
            pltpu.VMEM((CS * HIST, DP), jnp.float32),
            pltpu.VMEM((SPW, 16), jnp.float32),
            pltpu.VMEM((16,), jnp.float32),
            pltpu.VMEM((24,), jnp.float32),
            pltpu.SemaphoreType.DMA,
            pltpu.SemaphoreType.DMA,
            pltpu.SemaphoreType.DMA,
            pltpu.SemaphoreType.DMA,
        ],
    )(p, idx4, b_pad)
    return out16[:, :NCLS]


# speedup vs baseline: 1.1543x; 1.1543x over previous
import jax
import jax.numpy as jnp
from jax import lax
from jax.experimental import pallas as pl
from jax.experimental.pallas import tpu as pltpu
from jax.experimental.pallas import tpu_sc as plsc

VOCAB = 1000000
EMBED = 64
NCLS = 5
BATCH = 16384
HIST = 200

DP = 8
NC, NS = 2, 16
NW = NC * NS
SPW = BATCH // NW
CS = 16
NCHUNK = SPW // CS

RB = 32768
SLOTS = 128 // DP
SB = RB // SLOTS
SHIFT = 11
NBLK = -(-VOCAB // RB)
VP = NBLK * RB


def _proj_body(et_ref, w_ref, p_ref):
    lhs = jnp.concatenate(
        [et_ref[:, SB * m:SB * (m + 1)] for m in range(SLOTS)], axis=0)
    p_ref[...] = lax.dot_general(lhs, w_ref[...],
                                 dimension_numbers=(((0,), (0,)), ((), ())),
                                 preferred_element_type=jnp.float32)


def _project_table(embed_t, w_pad):
    return pl.pallas_call(
        _proj_body,
        grid=(NBLK,),
        in_specs=[
            pl.BlockSpec((EMBED, RB), lambda i: (0, i)),
            pl.BlockSpec((SLOTS * EMBED, 128), lambda i: (0, 0)),
        ],
        out_specs=pl.BlockSpec((RB // SLOTS, 128), lambda i: (i, 0)),
        out_shape=jax.ShapeDtypeStruct((VP // SLOTS, 128), jnp.float32),
    )(embed_t, w_pad)


def _sc_body(p_hbm, idx_hbm, b_hbm, out_hbm,
             idxb0, idxb1, rowsb0, rowsb1, outv, bvecb, foldb,
             semg0, semg1, semi0, semi1):
    wid = lax.axis_index("s") * NC + lax.axis_index("c")
    base = wid * SPW
    pltpu.sync_copy(b_hbm, bvecb)
    foldb[pl.ds(8, 16)] = jnp.zeros((16,), jnp.float32)
    bufs = ((idxb0, rowsb0, semg0, semi0), (idxb1, rowsb1, semg1, semi1))

    def stage_idx(c, idxb, semi):
        pltpu.async_copy(
            idx_hbm.at[pl.ds((base + c * CS) * HIST, CS * HIST)], idxb, semi)

    def wait_idx(c, idxb, semi):
        pltpu.make_async_copy(
            idx_hbm.at[pl.ds((base + c * CS) * HIST, CS * HIST)],
            idxb, semi).wait()

    def remap(idxb):
        def rbody(j, carry):
            sl = pl.ds(j * 16, 16)
            vv = idxb[sl]
            idxb[sl] = ((vv & jnp.int32(~(RB - 1)))
                        | ((vv & jnp.int32(SB - 1)) << 4)
                        | ((vv >> SHIFT) & (SLOTS - 1)))
            return carry
        lax.fori_loop(0, CS * HIST // 16, rbody, 0)

    def fire_gathers(idxb, rowsb, semg):
        for k in range(CS * HIST // 128):
            pltpu.async_copy(p_hbm.at[idxb.at[pl.ds(128 * k, 128)]],
                             rowsb.at[pl.ds(128 * k, 128)], semg)

    def drain_gathers(rowsb, semg):
        pltpu.make_async_copy(p_hbm.at[pl.ds(0, CS * HIST)], rowsb, semg).wait()

    def accumulate(c, rowsb):
        bv = bvecb[...]
        z = jnp.zeros((16,), jnp.float32)
        rowpat = (lax.iota(jnp.int32, 16) >> 3) & 1
        colpat = lax.iota(jnp.int32, 16) & 7
        for s in range(CS):
            def body(j, accs):
                r = HIST * s + 8 * j
                return tuple(
                    accs[t] + plsc.load_gather(
                        rowsb, [rowpat + (r + 2 * t), colpat])
                    for t in range(4))
            accs = lax.fori_loop(0, HIST // 8, body, (z,) * 4)
            tot = (accs[0] + accs[1]) + (accs[2] + accs[3])
            foldb[pl.ds(0, 16)] = tot
            outv[c * CS + s] = tot + foldb[pl.ds(8, 16)] + bv

    idxb, rowsb, semg, semi = bufs[0]
    stage_idx(0, idxb, semi)
    wait_idx(0, idxb, semi)
    remap(idxb)
    fire_gathers(idxb, rowsb, semg)
    stage_idx(1, bufs[1][0], bufs[1][3])
    wait_idx(1, bufs[1][0], bufs[1][3])
    remap(bufs[1][0])

    def half(c, b):
        idxb, rowsb, semg, semi = bufs[b]
        nidxb, nrowsb, nsemg, nsemi = bufs[1 - b]
        drain_gathers(rowsb, semg)

        @pl.when(c + 1 < NCHUNK)
        def _():
            fire_gathers(nidxb, nrowsb, nsemg)

        @pl.when(c + 2 < NCHUNK)
        def _():
            stage_idx(c + 2, idxb, semi)

        accumulate(c, rowsb)

        @pl.when(c + 2 < NCHUNK)
        def _():
            wait_idx(c + 2, idxb, semi)
            remap(idxb)

    def pair(cc, carry):
        half(2 * cc, 0)
        half(2 * cc + 1, 1)
        return carry

    lax.fori_loop(0, NCHUNK // 2, pair, 0)
    pltpu.sync_copy(outv, out_hbm.at[pl.ds(base, SPW)])


@jax.jit
def kernel(inputs, embed_table, W, b):
    b_pad = jnp.zeros((16,), jnp.float32).at[:NCLS].set(b)
    w_bd = jnp.zeros((SLOTS * EMBED, 128), jnp.float32)
    for _m in range(SLOTS):
        w_bd = w_bd.at[EMBED * _m:EMBED * (_m + 1),
                       DP * _m:DP * _m + NCLS].set(W)
    p = jnp.reshape(_project_table(embed_table.T, w_bd), (VP, DP))
    idx1 = jnp.reshape(inputs.astype(jnp.int32), (BATCH * HIST,))

    mesh = plsc.VectorSubcoreMesh(core_axis_name="c", subcore_axis_name="s")
    out16 = pl.kernel(
        _sc_body,
        out_type=jax.ShapeDtypeStruct((BATCH, 16), jnp.float32),
        mesh=mesh,
        compiler_params=pltpu.CompilerParams(use_tc_tiling_on_sc=False,
                                             needs_layout_passes=False),
        scratch_types=[
            pltpu.VMEM((CS * HIST,), jnp.int32),
            pltpu.VMEM((CS * HIST,), jnp.int32),
            pltpu.VMEM((CS * HIST, DP), jnp.float32),
            pltpu.VMEM((CS * HIST, DP), jnp.float32),
            pltpu.VMEM((SPW, 16), jnp.float32),
            pltpu.VMEM((16,), jnp.float32),
            pltpu.VMEM((24,), jnp.float32),
            pltpu.SemaphoreType.DMA,
            pltpu.SemaphoreType.DMA,
            pltpu.SemaphoreType.DMA,
            pltpu.SemaphoreType.DMA,
        ],
    )(p, idx1, b_pad)
    return out16[:, :NCLS]
